# baseline (speedup 1.0000x reference)
import functools

import jax
import jax.numpy as jnp
import numpy as np
from jax import lax
from jax.experimental import pallas as pl
from jax.experimental.pallas import tpu as pltpu
from jax.experimental.pallas import tpu_sc as plsc

N = 10000
E = 320000
D = 128
H = 8
M = 8
HF = D // H
DA = D + 16
NC = 2
NS = 16
NW = NC * NS
B = 32
BLOCKS = E // B
MAXIT = -(-BLOCKS // NW)
MAXI2 = (MAXIT + 1) // 2
ROWS_PER_TILE = N // NS
ZR = 25
RT = 2000


def _proj_body(xs_ref, xd_ref, wq_ref, wk_ref, wt_ref, wv_ref,
               q_ref, k_ref, t_ref, v_ref):
    hp = lax.Precision.HIGHEST
    xs = xs_ref[...]
    xd = xd_ref[...]
    q_ref[...] = jnp.dot(xd, wq_ref[...], precision=hp)
    k_ref[...] = jnp.dot(xs, wk_ref[...], precision=hp)
    t_ref[...] = jnp.dot(xs, wt_ref[...], precision=hp)
    v_ref[...] = jnp.dot(xs, wv_ref[...], precision=hp)


def _tc_proj(xs2, xd2, Wq, Wk, Wt, Wv):
    g = N // RT
    row = pl.BlockSpec((RT, D), lambda i: (i, 0))
    wsp = pl.BlockSpec((D, D), lambda i: (0, 0))
    return pl.pallas_call(
        _proj_body,
        grid=(g,),
        in_specs=[row, row, wsp, wsp, wsp, wsp],
        out_specs=[row, row, row, row],
        out_shape=[jax.ShapeDtypeStruct((N, D), jnp.float32),
                   jax.ShapeDtypeStruct((N, D), jnp.float32),
                   jax.ShapeDtypeStruct((N, D), jnp.float32),
                   jax.ShapeDtypeStruct((N, D), jnp.float32)],
    )(xs2, xd2, Wq, Wk, Wt, Wv)


def _node_body(h0_ref, h1_ref, t_ref, wi_ref, w2_ref, bi_ref, se_ref,
               fi_ref, rden_ref):
    hp = lax.Precision.HIGHEST
    acc = h0_ref[...] + h1_ref[...]
    hu = acc[:, :D]
    den = acc[:, D:D + H]
    den = jnp.where(den == 0.0, 1.0, den)
    lanes = lax.broadcasted_iota(jnp.int32, (H, D), 1)
    heads = lax.broadcasted_iota(jnp.int32, (H, D), 0)
    sel = jnp.where(lanes // HF == heads, 1.0, 0.0)
    rden = 1.0 / jnp.dot(den, sel, precision=hp)
    rden_ref[...] = rden
    hmat = hu * rden
    wi = wi_ref[...]
    wi16 = wi[:HF, :]
    tw = t_ref[...] * wi[HF:HF + 1, :] + bi_ref[...]
    se = jnp.exp(se_ref[...])
    outs = []
    for h in range(H):
        seg = hmat[:, h * HF:(h + 1) * HF]
        mu = jax.nn.sigmoid(jnp.dot(seg, wi16, precision=hp) + tw)
        z = jnp.dot(mu, w2_ref[...], precision=hp) / se
        outs.append(se * jnp.log1p(jnp.exp(z)))
    fi_ref[...] = jnp.concatenate(outs, axis=1)


def _tc_node(huv0, huv1, t2, Wi, W2, bi2, se2):
    g = N // RT
    return pl.pallas_call(
        _node_body,
        grid=(g,),
        in_specs=[pl.BlockSpec((RT, DA), lambda i: (i, 0)),
                  pl.BlockSpec((RT, DA), lambda i: (i, 0)),
                  pl.BlockSpec((RT, 1), lambda i: (i, 0)),
                  pl.BlockSpec((HF + 1, D), lambda i: (0, 0)),
                  pl.BlockSpec((D, M), lambda i: (0, 0)),
                  pl.BlockSpec((1, D), lambda i: (0, 0)),
                  pl.BlockSpec((1, M), lambda i: (0, 0))],
        out_specs=[pl.BlockSpec((RT, H * M), lambda i: (i, 0)),
                   pl.BlockSpec((RT, D), lambda i: (i, 0))],
        out_shape=[jax.ShapeDtypeStruct((N, H * M), jnp.float32),
                   jax.ShapeDtypeStruct((N, D), jnp.float32)],
    )(huv0, huv1, t2, Wi, W2, bi2, se2)


def _final_body(o0_ref, o1_ref, rden_ref, xd_ref, out_ref):
    out_ref[...] = (o0_ref[...] + o1_ref[...]) * rden_ref[...] + xd_ref[...]


def _tc_final(ou0, ou1, rden, xd2):
    g = N // RT
    row = pl.BlockSpec((RT, D), lambda i: (i, 0))
    return pl.pallas_call(
        _final_body,
        grid=(g,),
        in_specs=[row, row, row, row],
        out_specs=row,
        out_shape=jax.ShapeDtypeStruct((N, D), jnp.float32),
    )(ou0, ou1, rden, xd2)


_MESH = plsc.VectorSubcoreMesh(core_axis_name="c", subcore_axis_name="s",
                               num_cores=NC, num_subcores=NS)
_SC_PARAMS = pltpu.CompilerParams(needs_layout_passes=False,
                                  use_tc_tiling_on_sc=False)
_ISCALE = float(1.0 / np.sqrt(HF))


def _zero_vmem(buf, rows, width):
    def zrow(r, _):
        for c in range(width // 16):
            buf[r, pl.ds(c * 16, 16)] = jnp.zeros((16,), jnp.float32)
        return 0
    lax.fori_loop(0, rows, zrow, 0)


def _zero_shared(shared, stage, sid):
    row0 = sid * ROWS_PER_TILE

    def zcp(k, _):
        pltpu.sync_copy(stage.at[pl.ds(0, ZR)],
                        shared.at[pl.ds(row0 + k * ZR, ZR)])
        return 0
    lax.fori_loop(0, ROWS_PER_TILE // ZR, zcp, 0)


def _flush_shared(shared, stage, out_hbm, cid, sid):
    def fl(k, _):
        r0 = sid * ROWS_PER_TILE + k * ZR
        pltpu.sync_copy(shared.at[pl.ds(r0, ZR)], stage.at[pl.ds(0, ZR)])
        pltpu.sync_copy(stage.at[pl.ds(0, ZR)], out_hbm.at[cid, pl.ds(r0, ZR)])
        return 0
    lax.fori_loop(0, ROWS_PER_TILE // ZR, fl, 0)


@functools.partial(
    pl.kernel,
    out_type=(jax.ShapeDtypeStruct((E, H), jnp.float32),
              jax.ShapeDtypeStruct((NC, N, DA), jnp.float32)),
    mesh=_MESH,
    compiler_params=_SC_PARAMS,
    scratch_types=[
        pltpu.VMEM((2, 3, B), jnp.int32),
        pltpu.VMEM((2, B), jnp.int32),
        pltpu.VMEM((D,), jnp.float32),
        pltpu.VMEM((2, B, D), jnp.float32),
        pltpu.VMEM((2, B, D), jnp.float32),
        pltpu.VMEM((2, B, D), jnp.float32),
        pltpu.VMEM((2, B, DA), jnp.float32),
        pltpu.VMEM((2, B, H), jnp.float32),
        pltpu.VMEM_SHARED((N, DA), jnp.float32),
        pltpu.SemaphoreType.DMA,
        pltpu.SemaphoreType.DMA,
        pltpu.SemaphoreType.DMA,
        pltpu.SemaphoreType.DMA,
        pltpu.SemaphoreType.DMA,
        pltpu.SemaphoreType.DMA,
        pltpu.SemaphoreType.DMA,
        pltpu.SemaphoreType.DMA,
    ],
)
def _sc_pass1(q_hbm, k_hbm, t_hbm, we_hbm, meta_hbm,
              a_hbm, huv_hbm,
              mbuf, didxs, webuf, qbuf, kbuf, tbuf, tabuf, abuf,
              hu_s,
              semm0, semm1, semg0, semg1, semsc0, semsc1, sema0, sema1):
    cid = lax.axis_index("c")
    sid = lax.axis_index("s")
    wid = cid * NS + sid
    semm = (semm0, semm1)
    semg = (semg0, semg1)
    semsc = (semsc0, semsc1)
    sema = (sema0, sema1)

    _zero_vmem(tabuf.at[0], B, DA)
    _zero_vmem(tabuf.at[1], B, DA)
    _zero_shared(hu_s, tabuf.at[0], sid)
    pltpu.sync_copy(we_hbm, webuf)
    plsc.subcore_barrier()

    def blk_of(it):
        return wid + it * NW

    def issue(it, p):
        blk = blk_of(it)

        @pl.when(blk < BLOCKS)
        def _():
            mb = mbuf.at[p]
            pltpu.make_async_copy(meta_hbm.at[blk], mb, semm[p]).wait()

    def process(it, p):
        blk = blk_of(it)

        @pl.when(blk < BLOCKS)
        def _():
            mb = mbuf.at[p]
            tab = tabuf.at[p]
            ab = abuf.at[p]
            base = blk * B

            @pl.when(it >= 2)
            def _():
                pltpu.make_async_copy(ab, a_hbm.at[pl.ds(base, B)], sema[p]).wait()

            for c in range(B // 16):
                didxs[p, pl.ds(c * 16, 16)] = mb[1, pl.ds(c * 16, 16)]

            def group(g, _):
                rows = g * 16 + lax.iota(jnp.int32, 16)
                efv = plsc.bitcast(mb[2, pl.ds(g * 16, 16)], jnp.float32)
                qb = qbuf.at[p]
                kb = kbuf.at[p]
                tb = tbuf.at[p]
                for h in range(H):
                    acc = jnp.zeros((16,), jnp.float32)
                    accw = jnp.zeros((16,), jnp.float32)
                    for f in range(HF):
                        col = jnp.full((16,), h * HF + f, jnp.int32)
                        qg = plsc.load_gather(qb, [rows, col])
                        kg = plsc.load_gather(kb, [rows, col])
                        weg = plsc.load_gather(webuf, [col])
                        acc = acc + kg * qg
                        accw = accw + weg * qg
                    ah = jnp.exp((acc + efv * accw) * _ISCALE)
                    plsc.store_scatter(ab, [rows, jnp.full((16,), h, jnp.int32)], ah)
                    plsc.store_scatter(tab, [rows, jnp.full((16,), D + h, jnp.int32)], ah)
                    for f in range(HF):
                        col = jnp.full((16,), h * HF + f, jnp.int32)
                        tg = plsc.load_gather(tb, [rows, col])
                        plsc.store_scatter(tab, [rows, col], tg * ah)
                return 0

            lax.fori_loop(0, B // 16, group, 0)
            pltpu.async_copy(ab, a_hbm.at[pl.ds(base, B)], sema[p])

            blk2 = blk + 2 * NW

            @pl.when(blk2 < BLOCKS)
            def _():
                pltpu.async_copy(meta_hbm.at[blk2], mb, semm[p])

    pltpu.sync_copy(meta_hbm.at[blk_of(0)], mbuf.at[0])
    pltpu.async_copy(meta_hbm.at[blk_of(1)], mbuf.at[1], semm1)


    def body2(k, _):
        issue(2 * k + 1, 1)
        process(2 * k, 0)
        issue(2 * k + 2, 0)
        process(2 * k + 1, 1)
        return 0

    lax.fori_loop(0, MAXI2, body2, 0)

    pltpu.make_async_copy(abuf.at[0], a_hbm.at[pl.ds(0, B)], sema0).wait()
    pltpu.make_async_copy(abuf.at[1], a_hbm.at[pl.ds(0, B)], sema1).wait()
    plsc.subcore_barrier()
    _flush_shared(hu_s, tabuf.at[0], huv_hbm, cid, sid)


@functools.partial(
    pl.kernel,
    out_type=jax.ShapeDtypeStruct((NC, N, D), jnp.float32),
    mesh=_MESH,
    compiler_params=_SC_PARAMS,
    scratch_types=[
        pltpu.VMEM((2, 3, B), jnp.int32),
        pltpu.VMEM((2, B), jnp.int32),
        pltpu.VMEM((2, B, H * M), jnp.float32),
        pltpu.VMEM((2, B, H * M), jnp.float32),
        pltpu.VMEM((2, B, D), jnp.float32),
        pltpu.VMEM((2, B, D), jnp.float32),
        pltpu.VMEM((2, B, H), jnp.float32),
        pltpu.VMEM_SHARED((N, D), jnp.float32),
        pltpu.SemaphoreType.DMA,
        pltpu.SemaphoreType.DMA,
        pltpu.SemaphoreType.DMA,
        pltpu.SemaphoreType.DMA,
        pltpu.SemaphoreType.DMA,
        pltpu.SemaphoreType.DMA,
    ],
)
def _sc_pass2(fi_hbm, m_hbm, v_hbm, a_hbm, meta_hbm,
              ou_hbm,
              mbuf, didxs, fibuf, m2buf, vbuf, obuf, ain,
              ou_s,
              semm0, semm1, semg0, semg1, semsc0, semsc1):
    cid = lax.axis_index("c")
    sid = lax.axis_index("s")
    wid = cid * NS + sid
    semm = (semm0, semm1)
    semg = (semg0, semg1)
    semsc = (semsc0, semsc1)

    _zero_vmem(obuf.at[0], B, D)
    _zero_shared(ou_s, obuf.at[0], sid)
    plsc.subcore_barrier()

    def blk_of(it):
        return wid + it * NW

    def issue(it, p):
        blk = blk_of(it)

        @pl.when(blk < BLOCKS)
        def _():
            mb = mbuf.at[p]
            base = blk * B
            pltpu.make_async_copy(meta_hbm.at[blk], mb, semm[p]).wait()
            pltpu.async_copy(fi_hbm.at[mb.at[0]], fibuf.at[p], semg[p])
            pltpu.async_copy(m_hbm.at[mb.at[1]], m2buf.at[p], semg[p])
            pltpu.async_copy(v_hbm.at[mb.at[0]], vbuf.at[p], semg[p])
            pltpu.async_copy(a_hbm.at[pl.ds(base, B)], ain.at[p], semg[p])

    def process(it, p):
        blk = blk_of(it)

        @pl.when(blk < BLOCKS)
        def _():
            mb = mbuf.at[p]
            ob = obuf.at[p]
            base = blk * B
            pltpu.make_async_copy(fi_hbm.at[mb.at[0]], fibuf.at[p], semg[p]).wait()
            pltpu.make_async_copy(m_hbm.at[mb.at[1]], m2buf.at[p], semg[p]).wait()
            pltpu.make_async_copy(v_hbm.at[mb.at[0]], vbuf.at[p], semg[p]).wait()
            pltpu.make_async_copy(a_hbm.at[pl.ds(base, B)], ain.at[p], semg[p]).wait()

            @pl.when(it >= 2)
            def _():
                pltpu.make_async_copy(ob, ou_s.at[didxs.at[p]], semsc[p]).wait()

            for c in range(B // 16):
                didxs[p, pl.ds(c * 16, 16)] = mb[1, pl.ds(c * 16, 16)]

            def group(g, _):
                rows = g * 16 + lax.iota(jnp.int32, 16)
                fb = fibuf.at[p]
                m2 = m2buf.at[p]
                vb = vbuf.at[p]
                ai = ain.at[p]
                for h in range(H):
                    acc = jnp.zeros((16,), jnp.float32)
                    for j in range(M):
                        col = jnp.full((16,), h * M + j, jnp.int32)
                        fg = plsc.load_gather(fb, [rows, col])
                        mg = plsc.load_gather(m2, [rows, col])
                        acc = acc + fg * mg
                    ag = plsc.load_gather(ai, [rows, jnp.full((16,), h, jnp.int32)])
                    ee = acc * ag
                    for f in range(HF):
                        col = jnp.full((16,), h * HF + f, jnp.int32)
                        vg = plsc.load_gather(vb, [rows, col])
                        plsc.store_scatter(ob, [rows, col], vg * ee)
                return 0

            lax.fori_loop(0, B // 16, group, 0)
            pltpu.async_copy(ob, ou_s.at[didxs.at[p]], semsc[p], add=True)

            blk2 = blk + 2 * NW

            @pl.when(blk2 < BLOCKS)
            def _():
                pltpu.async_copy(meta_hbm.at[blk2], mb, semm[p])

    pltpu.sync_copy(meta_hbm.at[blk_of(0)], mbuf.at[0])
    pltpu.async_copy(meta_hbm.at[blk_of(1)], mbuf.at[1], semm1)
    pltpu.async_copy(fi_hbm.at[mbuf.at[0].at[0]], fibuf.at[0], semg0)
    pltpu.async_copy(m_hbm.at[mbuf.at[0].at[1]], m2buf.at[0], semg0)
    pltpu.async_copy(v_hbm.at[mbuf.at[0].at[0]], vbuf.at[0], semg0)
    pltpu.async_copy(a_hbm.at[pl.ds(wid * B, B)], ain.at[0], semg0)

    def body2(k, _):
        issue(2 * k + 1, 1)
        process(2 * k, 0)
        issue(2 * k + 2, 0)
        process(2 * k + 1, 1)
        return 0

    lax.fori_loop(0, MAXI2, body2, 0)

    pltpu.make_async_copy(obuf.at[0], ou_s.at[didxs.at[0]], semsc0).wait()
    pltpu.make_async_copy(obuf.at[1], ou_s.at[didxs.at[1]], semsc1).wait()
    plsc.subcore_barrier()
    _flush_shared(ou_s, obuf.at[0], ou_hbm, cid, sid)


def kernel(x_src, x_dst, t, m, ef, Wq, Wk, Wv, Wt, We, Wi, bi,
           weight_i, scale_i, edge_index):
    xs2 = x_src.reshape(N, D)
    xd2 = x_dst.reshape(N, D)
    src = edge_index[0]
    dst = edge_index[1]
    ef_bits = lax.bitcast_convert_type(ef.reshape(E), jnp.int32)

    meta = jnp.stack([src.reshape(BLOCKS, B), dst.reshape(BLOCKS, B),
                      ef_bits.reshape(BLOCKS, B)], axis=1)

    Q, K, T, V = _tc_proj(xs2, xd2, Wq, Wk, Wt, Wv)
    a_e, huv = _sc_pass1(Q, K, T, We.reshape(D), meta)

    flat = weight_i.reshape(HF * M)
    sel = jnp.arange(D)[:, None] // HF == jnp.arange(M)[None, :]
    W2 = jnp.where(sel, flat[:, None], 0.0)
    fi64, rden = _tc_node(huv[0], huv[1], t.reshape(N, 1), Wi, W2,
                          bi.reshape(1, HF * M), scale_i.reshape(1, M))

    ou = _sc_pass2(fi64, m.reshape(N, H * M), V, a_e, meta)
    out2 = _tc_final(ou[0], ou[1], rden, xd2)
    return out2.reshape(N, 1, D)


# speedup vs baseline: 3.1041x; 3.1041x over previous
import functools

import jax
import jax.numpy as jnp
import numpy as np
from jax import lax
from jax.experimental import pallas as pl
from jax.experimental.pallas import tpu as pltpu
from jax.experimental.pallas import tpu_sc as plsc

N = 10000
E = 320000
D = 128
H = 8
M = 8
HF = D // H
DA = D + 16
DQ = D + H
NC = 2
NS = 16
NW = NC * NS
B = 32
BLOCKS = E // B
MAXIT = -(-BLOCKS // NW)
MAXI2 = (MAXIT + 1) // 2
ROWS_PER_TILE = N // NS
ZR = 25
RT = 2000


def _proj_body(xs_ref, xd_ref, wq_ref, wk_ref, wt_ref, wv_ref, we_ref,
               q_ref, k_ref, t_ref, v_ref):
    hp = lax.Precision.HIGHEST
    xs = xs_ref[...]
    xd = xd_ref[...]
    q = jnp.dot(xd, wq_ref[...], precision=hp)
    lanes = lax.broadcasted_iota(jnp.int32, (D, H), 0)
    heads = lax.broadcasted_iota(jnp.int32, (D, H), 1)
    sel = jnp.where(lanes // HF == heads, 1.0, 0.0)
    wq8 = jnp.dot(q * we_ref[...], sel, precision=hp)
    q_ref[...] = jnp.concatenate([q, wq8], axis=1)
    k_ref[...] = jnp.dot(xs, wk_ref[...], precision=hp)
    t_ref[...] = jnp.dot(xs, wt_ref[...], precision=hp)
    v_ref[...] = jnp.dot(xs, wv_ref[...], precision=hp)


def _tc_proj(xs2, xd2, Wq, Wk, Wt, Wv, We):
    g = N // RT
    row = pl.BlockSpec((RT, D), lambda i: (i, 0))
    wsp = pl.BlockSpec((D, D), lambda i: (0, 0))
    return pl.pallas_call(
        _proj_body,
        grid=(g,),
        in_specs=[row, row, wsp, wsp, wsp, wsp,
                  pl.BlockSpec((1, D), lambda i: (0, 0))],
        out_specs=[pl.BlockSpec((RT, DQ), lambda i: (i, 0)), row, row, row],
        out_shape=[jax.ShapeDtypeStruct((N, DQ), jnp.float32),
                   jax.ShapeDtypeStruct((N, D), jnp.float32),
                   jax.ShapeDtypeStruct((N, D), jnp.float32),
                   jax.ShapeDtypeStruct((N, D), jnp.float32)],
    )(xs2, xd2, Wq, Wk, Wt, Wv, We)


def _node_body(h0_ref, h1_ref, t_ref, wi_ref, w2_ref, bi_ref, se_ref,
               fi_ref, rden_ref):
    hp = lax.Precision.HIGHEST
    acc = h0_ref[...] + h1_ref[...]
    hu = acc[:, :D]
    den = acc[:, D:D + H]
    den = jnp.where(den == 0.0, 1.0, den)
    lanes = lax.broadcasted_iota(jnp.int32, (H, D), 1)
    heads = lax.broadcasted_iota(jnp.int32, (H, D), 0)
    sel = jnp.where(lanes // HF == heads, 1.0, 0.0)
    rden = 1.0 / jnp.dot(den, sel, precision=hp)
    rden_ref[...] = rden
    hmat = hu * rden
    wi = wi_ref[...]
    wi16 = wi[:HF, :]
    tw = t_ref[...] * wi[HF:HF + 1, :] + bi_ref[...]
    se = jnp.exp(se_ref[...])
    outs = []
    for h in range(H):
        seg = hmat[:, h * HF:(h + 1) * HF]
        mu = jax.nn.sigmoid(jnp.dot(seg, wi16, precision=hp) + tw)
        z = jnp.dot(mu, w2_ref[...], precision=hp) / se
        outs.append(se * jnp.log1p(jnp.exp(z)))
    fi_ref[...] = jnp.concatenate(outs, axis=1)


def _tc_node(huv0, huv1, t2, Wi, W2, bi2, se2):
    g = N // RT
    return pl.pallas_call(
        _node_body,
        grid=(g,),
        in_specs=[pl.BlockSpec((RT, DA), lambda i: (i, 0)),
                  pl.BlockSpec((RT, DA), lambda i: (i, 0)),
                  pl.BlockSpec((RT, 1), lambda i: (i, 0)),
                  pl.BlockSpec((HF + 1, D), lambda i: (0, 0)),
                  pl.BlockSpec((D, M), lambda i: (0, 0)),
                  pl.BlockSpec((1, D), lambda i: (0, 0)),
                  pl.BlockSpec((1, M), lambda i: (0, 0))],
        out_specs=[pl.BlockSpec((RT, H * M), lambda i: (i, 0)),
                   pl.BlockSpec((RT, D), lambda i: (i, 0))],
        out_shape=[jax.ShapeDtypeStruct((N, H * M), jnp.float32),
                   jax.ShapeDtypeStruct((N, D), jnp.float32)],
    )(huv0, huv1, t2, Wi, W2, bi2, se2)


def _final_body(o0_ref, o1_ref, rden_ref, xd_ref, out_ref):
    out_ref[...] = (o0_ref[...] + o1_ref[...]) * rden_ref[...] + xd_ref[...]


def _tc_final(ou0, ou1, rden, xd2):
    g = N // RT
    row = pl.BlockSpec((RT, D), lambda i: (i, 0))
    return pl.pallas_call(
        _final_body,
        grid=(g,),
        in_specs=[row, row, row, row],
        out_specs=row,
        out_shape=jax.ShapeDtypeStruct((N, D), jnp.float32),
    )(ou0, ou1, rden, xd2)


_MESH = plsc.VectorSubcoreMesh(core_axis_name="c", subcore_axis_name="s",
                               num_cores=NC, num_subcores=NS)
_SC_PARAMS = pltpu.CompilerParams(needs_layout_passes=False,
                                  use_tc_tiling_on_sc=False)
_ISCALE = float(1.0 / np.sqrt(HF))


def _zero_vmem(buf, rows, width):
    def zrow(r, _):
        for c in range(width // 16):
            buf[r, pl.ds(c * 16, 16)] = jnp.zeros((16,), jnp.float32)
        return 0
    lax.fori_loop(0, rows, zrow, 0)


def _zero_shared(shared, stage, sid):
    row0 = sid * ROWS_PER_TILE

    def zcp(k, _):
        pltpu.sync_copy(stage.at[pl.ds(0, ZR)],
                        shared.at[pl.ds(row0 + k * ZR, ZR)])
        return 0
    lax.fori_loop(0, ROWS_PER_TILE // ZR, zcp, 0)


def _flush_shared(shared, stage, out_hbm, cid, sid):
    def fl(k, _):
        r0 = sid * ROWS_PER_TILE + k * ZR
        pltpu.sync_copy(shared.at[pl.ds(r0, ZR)], stage.at[pl.ds(0, ZR)])
        pltpu.sync_copy(stage.at[pl.ds(0, ZR)], out_hbm.at[cid, pl.ds(r0, ZR)])
        return 0
    lax.fori_loop(0, ROWS_PER_TILE // ZR, fl, 0)


@functools.partial(
    pl.kernel,
    out_type=(jax.ShapeDtypeStruct((E, H), jnp.float32),
              jax.ShapeDtypeStruct((NC, N, DA), jnp.float32)),
    mesh=_MESH,
    compiler_params=_SC_PARAMS,
    scratch_types=[
        pltpu.VMEM((2, 3, B), jnp.int32),
        pltpu.VMEM((2, B), jnp.int32),
        pltpu.VMEM((2, B, DQ), jnp.float32),
        pltpu.VMEM((2, B, D), jnp.float32),
        pltpu.VMEM((2, B, D), jnp.float32),
        pltpu.VMEM((2, B, DA), jnp.float32),
        pltpu.VMEM((2, B, H), jnp.float32),
        pltpu.VMEM_SHARED((N, DA), jnp.float32),
        pltpu.SemaphoreType.DMA,
        pltpu.SemaphoreType.DMA,
        pltpu.SemaphoreType.DMA,
        pltpu.SemaphoreType.DMA,
        pltpu.SemaphoreType.DMA,
        pltpu.SemaphoreType.DMA,
        pltpu.SemaphoreType.DMA,
        pltpu.SemaphoreType.DMA,
    ],
)
def _sc_pass1(q_hbm, k_hbm, t_hbm, meta_hbm,
              a_hbm, huv_hbm,
              mbuf, didxs, qbuf, kbuf, tbuf, tabuf, abuf,
              hu_s,
              semm0, semm1, semg0, semg1, semsc0, semsc1, sema0, sema1):
    cid = lax.axis_index("c")
    sid = lax.axis_index("s")
    wid = cid * NS + sid
    semm = (semm0, semm1)
    semg = (semg0, semg1)
    semsc = (semsc0, semsc1)
    sema = (sema0, sema1)

    _zero_vmem(tabuf.at[0], B, DA)
    _zero_vmem(tabuf.at[1], B, DA)
    _zero_shared(hu_s, tabuf.at[0], sid)
    plsc.subcore_barrier()

    def blk_of(it):
        return wid + it * NW

    def issue(it, p):
        blk = blk_of(it)

        @pl.when(blk < BLOCKS)
        def _():
            mb = mbuf.at[p]
            pltpu.make_async_copy(meta_hbm.at[blk], mb, semm[p]).wait()
            pltpu.async_copy(q_hbm.at[mb.at[1]], qbuf.at[p], semg[p])
            pltpu.async_copy(k_hbm.at[mb.at[0]], kbuf.at[p], semg[p])
            pltpu.async_copy(t_hbm.at[mb.at[0]], tbuf.at[p], semg[p])

    def process(it, p):
        blk = blk_of(it)

        @pl.when(blk < BLOCKS)
        def _():
            mb = mbuf.at[p]
            tab = tabuf.at[p]
            ab = abuf.at[p]
            base = blk * B
            pltpu.make_async_copy(q_hbm.at[mb.at[1]], qbuf.at[p], semg[p]).wait()
            pltpu.make_async_copy(k_hbm.at[mb.at[0]], kbuf.at[p], semg[p]).wait()
            pltpu.make_async_copy(t_hbm.at[mb.at[0]], tbuf.at[p], semg[p]).wait()

            @pl.when(it >= 2)
            def _():
                pltpu.make_async_copy(tab, hu_s.at[didxs.at[p]], semsc[p]).wait()
                pltpu.make_async_copy(ab, a_hbm.at[pl.ds(base, B)], sema[p]).wait()

            for c in range(B // 16):
                didxs[p, pl.ds(c * 16, 16)] = mb[1, pl.ds(c * 16, 16)]

            def group(g, _):
                rot = lax.iota(jnp.int32, 16)
                rows = g * 16 + rot
                efv = plsc.bitcast(mb[2, pl.ds(g * 16, 16)], jnp.float32)
                qb = qbuf.at[p]
                kb = kbuf.at[p]
                tb = tbuf.at[p]
                def dot_step(f, accs):
                    rc = jnp.bitwise_and(f + rot, HF - 1)
                    out = []
                    for h in range(H):
                        col = rc + h * HF
                        qg = plsc.load_gather(qb, [rows, col])
                        kg = plsc.load_gather(kb, [rows, col])
                        out.append(accs[h] + kg * qg)
                    return tuple(out)

                zero16 = jnp.zeros((16,), jnp.float32)
                accs = lax.fori_loop(0, HF, dot_step, (zero16,) * H)
                ahs = []
                for h in range(H):
                    wqg = plsc.load_gather(qb, [rows, jnp.full((16,), D + h, jnp.int32)])
                    ah = jnp.exp((accs[h] + efv * wqg) * _ISCALE)
                    plsc.store_scatter(ab, [rows, jnp.full((16,), h, jnp.int32)], ah)
                    plsc.store_scatter(tab, [rows, jnp.full((16,), D + h, jnp.int32)], ah)
                    ahs.append(ah)

                def tsc_step(f, c):
                    rc = jnp.bitwise_and(f + rot, HF - 1)
                    for h in range(H):
                        col = rc + h * HF
                        tg = plsc.load_gather(tb, [rows, col])
                        plsc.store_scatter(tab, [rows, col], tg * ahs[h])
                    return c

                lax.fori_loop(0, HF, tsc_step, 0)
                return 0

            lax.fori_loop(0, B // 16, group, 0)
            pltpu.async_copy(ab, a_hbm.at[pl.ds(base, B)], sema[p])
            pltpu.async_copy(tab, hu_s.at[didxs.at[p]], semsc[p], add=True)

            blk2 = blk + 2 * NW

            @pl.when(blk2 < BLOCKS)
            def _():
                pltpu.async_copy(meta_hbm.at[blk2], mb, semm[p])

    pltpu.sync_copy(meta_hbm.at[blk_of(0)], mbuf.at[0])
    pltpu.async_copy(meta_hbm.at[blk_of(1)], mbuf.at[1], semm1)
    pltpu.async_copy(q_hbm.at[mbuf.at[0].at[1]], qbuf.at[0], semg0)
    pltpu.async_copy(k_hbm.at[mbuf.at[0].at[0]], kbuf.at[0], semg0)
    pltpu.async_copy(t_hbm.at[mbuf.at[0].at[0]], tbuf.at[0], semg0)

    def body2(k, _):
        issue(2 * k + 1, 1)
        process(2 * k, 0)
        issue(2 * k + 2, 0)
        process(2 * k + 1, 1)
        return 0

    lax.fori_loop(0, MAXI2, body2, 0)

    pltpu.make_async_copy(tabuf.at[0], hu_s.at[didxs.at[0]], semsc0).wait()
    pltpu.make_async_copy(abuf.at[0], a_hbm.at[pl.ds(0, B)], sema0).wait()
    pltpu.make_async_copy(tabuf.at[1], hu_s.at[didxs.at[1]], semsc1).wait()
    pltpu.make_async_copy(abuf.at[1], a_hbm.at[pl.ds(0, B)], sema1).wait()
    plsc.subcore_barrier()
    _flush_shared(hu_s, tabuf.at[0], huv_hbm, cid, sid)


@functools.partial(
    pl.kernel,
    out_type=jax.ShapeDtypeStruct((NC, N, D), jnp.float32),
    mesh=_MESH,
    compiler_params=_SC_PARAMS,
    scratch_types=[
        pltpu.VMEM((2, 3, B), jnp.int32),
        pltpu.VMEM((2, B), jnp.int32),
        pltpu.VMEM((2, B, H * M), jnp.float32),
        pltpu.VMEM((2, B, H * M), jnp.float32),
        pltpu.VMEM((2, B, D), jnp.float32),
        pltpu.VMEM((2, B, D), jnp.float32),
        pltpu.VMEM((2, B, H), jnp.float32),
        pltpu.VMEM_SHARED((N, D), jnp.float32),
        pltpu.SemaphoreType.DMA,
        pltpu.SemaphoreType.DMA,
        pltpu.SemaphoreType.DMA,
        pltpu.SemaphoreType.DMA,
        pltpu.SemaphoreType.DMA,
        pltpu.SemaphoreType.DMA,
    ],
)
def _sc_pass2(fi_hbm, m_hbm, v_hbm, a_hbm, meta_hbm,
              ou_hbm,
              mbuf, didxs, fibuf, m2buf, vbuf, obuf, ain,
              ou_s,
              semm0, semm1, semg0, semg1, semsc0, semsc1):
    cid = lax.axis_index("c")
    sid = lax.axis_index("s")
    wid = cid * NS + sid
    semm = (semm0, semm1)
    semg = (semg0, semg1)
    semsc = (semsc0, semsc1)

    _zero_vmem(obuf.at[0], B, D)
    _zero_shared(ou_s, obuf.at[0], sid)
    plsc.subcore_barrier()

    def blk_of(it):
        return wid + it * NW

    def issue(it, p):
        blk = blk_of(it)

        @pl.when(blk < BLOCKS)
        def _():
            mb = mbuf.at[p]
            base = blk * B
            pltpu.make_async_copy(meta_hbm.at[blk], mb, semm[p]).wait()
            pltpu.async_copy(fi_hbm.at[mb.at[0]], fibuf.at[p], semg[p])
            pltpu.async_copy(m_hbm.at[mb.at[1]], m2buf.at[p], semg[p])
            pltpu.async_copy(v_hbm.at[mb.at[0]], vbuf.at[p], semg[p])
            pltpu.async_copy(a_hbm.at[pl.ds(base, B)], ain.at[p], semg[p])

    def process(it, p):
        blk = blk_of(it)

        @pl.when(blk < BLOCKS)
        def _():
            mb = mbuf.at[p]
            ob = obuf.at[p]
            base = blk * B
            pltpu.make_async_copy(fi_hbm.at[mb.at[0]], fibuf.at[p], semg[p]).wait()
            pltpu.make_async_copy(m_hbm.at[mb.at[1]], m2buf.at[p], semg[p]).wait()
            pltpu.make_async_copy(v_hbm.at[mb.at[0]], vbuf.at[p], semg[p]).wait()
            pltpu.make_async_copy(a_hbm.at[pl.ds(base, B)], ain.at[p], semg[p]).wait()

            @pl.when(it >= 2)
            def _():
                pltpu.make_async_copy(ob, ou_s.at[didxs.at[p]], semsc[p]).wait()

            for c in range(B // 16):
                didxs[p, pl.ds(c * 16, 16)] = mb[1, pl.ds(c * 16, 16)]

            def group(g, _):
                rot = lax.iota(jnp.int32, 16)
                rows = g * 16 + rot
                fb = fibuf.at[p]
                m2 = m2buf.at[p]
                vb = vbuf.at[p]
                ai = ain.at[p]
                def dot_step(j, accs):
                    rc = jnp.bitwise_and(j + rot, M - 1)
                    out = []
                    for h in range(H):
                        col = rc + h * M
                        fg = plsc.load_gather(fb, [rows, col])
                        mg = plsc.load_gather(m2, [rows, col])
                        out.append(accs[h] + fg * mg)
                    return tuple(out)

                zero16 = jnp.zeros((16,), jnp.float32)
                accs = lax.fori_loop(0, M, dot_step, (zero16,) * H)
                ees = []
                for h in range(H):
                    ag = plsc.load_gather(ai, [rows, jnp.full((16,), h, jnp.int32)])
                    ees.append(accs[h] * ag)

                def vsc_step(f, c):
                    rc = jnp.bitwise_and(f + rot, HF - 1)
                    for h in range(H):
                        col = rc + h * HF
                        vg = plsc.load_gather(vb, [rows, col])
                        plsc.store_scatter(ob, [rows, col], vg * ees[h])
                    return c

                lax.fori_loop(0, HF, vsc_step, 0)
                return 0

            lax.fori_loop(0, B // 16, group, 0)
            pltpu.async_copy(ob, ou_s.at[didxs.at[p]], semsc[p], add=True)

            blk2 = blk + 2 * NW

            @pl.when(blk2 < BLOCKS)
            def _():
                pltpu.async_copy(meta_hbm.at[blk2], mb, semm[p])

    pltpu.sync_copy(meta_hbm.at[blk_of(0)], mbuf.at[0])
    pltpu.async_copy(meta_hbm.at[blk_of(1)], mbuf.at[1], semm1)
    pltpu.async_copy(fi_hbm.at[mbuf.at[0].at[0]], fibuf.at[0], semg0)
    pltpu.async_copy(m_hbm.at[mbuf.at[0].at[1]], m2buf.at[0], semg0)
    pltpu.async_copy(v_hbm.at[mbuf.at[0].at[0]], vbuf.at[0], semg0)
    pltpu.async_copy(a_hbm.at[pl.ds(wid * B, B)], ain.at[0], semg0)

    def body2(k, _):
        issue(2 * k + 1, 1)
        process(2 * k, 0)
        issue(2 * k + 2, 0)
        process(2 * k + 1, 1)
        return 0

    lax.fori_loop(0, MAXI2, body2, 0)

    pltpu.make_async_copy(obuf.at[0], ou_s.at[didxs.at[0]], semsc0).wait()
    pltpu.make_async_copy(obuf.at[1], ou_s.at[didxs.at[1]], semsc1).wait()
    plsc.subcore_barrier()
    _flush_shared(ou_s, obuf.at[0], ou_hbm, cid, sid)


def kernel(x_src, x_dst, t, m, ef, Wq, Wk, Wv, Wt, We, Wi, bi,
           weight_i, scale_i, edge_index):
    xs2 = x_src.reshape(N, D)
    xd2 = x_dst.reshape(N, D)
    src = edge_index[0]
    dst = edge_index[1]
    ef_bits = lax.bitcast_convert_type(ef.reshape(E), jnp.int32)

    meta = jnp.stack([src.reshape(BLOCKS, B), dst.reshape(BLOCKS, B),
                      ef_bits.reshape(BLOCKS, B)], axis=1)

    Qa, K, T, V = _tc_proj(xs2, xd2, Wq, Wk, Wt, Wv, We)
    a_e, huv = _sc_pass1(Qa, K, T, meta)

    flat = weight_i.reshape(HF * M)
    sel = jnp.arange(D)[:, None] // HF == jnp.arange(M)[None, :]
    W2 = jnp.where(sel, flat[:, None], 0.0)
    fi64, rden = _tc_node(huv[0], huv[1], t.reshape(N, 1), Wi, W2,
                          bi.reshape(1, HF * M), scale_i.reshape(1, M))

    ou = _sc_pass2(fi64, m.reshape(N, H * M), V, a_e, meta)
    out2 = _tc_final(ou[0], ou[1], rden, xd2)
    return out2.reshape(N, 1, D)


# speedup vs baseline: 3.1045x; 1.0002x over previous
import functools

import jax
import jax.numpy as jnp
import numpy as np
from jax import lax
from jax.experimental import pallas as pl
from jax.experimental.pallas import tpu as pltpu
from jax.experimental.pallas import tpu_sc as plsc

N = 10000
E = 320000
D = 128
H = 8
M = 8
HF = D // H
DA = D + 16
DQ = D + H
NC = 2
NS = 16
NW = NC * NS
B = 32
BLOCKS = E // B
MAXIT = -(-BLOCKS // NW)
MAXI2 = (MAXIT + 1) // 2
ROWS_PER_TILE = N // NS
ZR = 25
RT = 2000


def _proj_body(xs_ref, xd_ref, wq_ref, wk_ref, wt_ref, wv_ref, we_ref,
               q_ref, k_ref, v_ref):
    hp = lax.Precision.HIGHEST
    xs = xs_ref[...]
    xd = xd_ref[...]
    q = jnp.dot(xd, wq_ref[...], precision=hp)
    lanes = lax.broadcasted_iota(jnp.int32, (D, H), 0)
    heads = lax.broadcasted_iota(jnp.int32, (D, H), 1)
    sel = jnp.where(lanes // HF == heads, 1.0, 0.0)
    wq8 = jnp.dot(q * we_ref[...], sel, precision=hp)
    q_ref[...] = jnp.concatenate([q, wq8], axis=1)
    k_ref[...] = jnp.concatenate(
        [jnp.dot(xs, wk_ref[...], precision=hp),
         jnp.dot(xs, wt_ref[...], precision=hp)], axis=1)
    v_ref[...] = jnp.dot(xs, wv_ref[...], precision=hp)


def _tc_proj(xs2, xd2, Wq, Wk, Wt, Wv, We):
    g = N // RT
    row = pl.BlockSpec((RT, D), lambda i: (i, 0))
    wsp = pl.BlockSpec((D, D), lambda i: (0, 0))
    return pl.pallas_call(
        _proj_body,
        grid=(g,),
        in_specs=[row, row, wsp, wsp, wsp, wsp,
                  pl.BlockSpec((1, D), lambda i: (0, 0))],
        out_specs=[pl.BlockSpec((RT, DQ), lambda i: (i, 0)),
                   pl.BlockSpec((RT, 2 * D), lambda i: (i, 0)), row],
        out_shape=[jax.ShapeDtypeStruct((N, DQ), jnp.float32),
                   jax.ShapeDtypeStruct((N, 2 * D), jnp.float32),
                   jax.ShapeDtypeStruct((N, D), jnp.float32)],
    )(xs2, xd2, Wq, Wk, Wt, Wv, We)


def _node_body(h0_ref, h1_ref, t_ref, wi_ref, w2_ref, bi_ref, se_ref, v_ref,
               vf_ref, rden_ref):
    hp = lax.Precision.HIGHEST
    acc = h0_ref[...] + h1_ref[...]
    hu = acc[:, :D]
    den = acc[:, D:D + H]
    den = jnp.where(den == 0.0, 1.0, den)
    lanes = lax.broadcasted_iota(jnp.int32, (H, D), 1)
    heads = lax.broadcasted_iota(jnp.int32, (H, D), 0)
    sel = jnp.where(lanes // HF == heads, 1.0, 0.0)
    rden = 1.0 / jnp.dot(den, sel, precision=hp)
    rden_ref[...] = rden
    hmat = hu * rden
    wi = wi_ref[...]
    wi16 = wi[:HF, :]
    tw = t_ref[...] * wi[HF:HF + 1, :] + bi_ref[...]
    se = jnp.exp(se_ref[...])
    outs = []
    for h in range(H):
        seg = hmat[:, h * HF:(h + 1) * HF]
        mu = jax.nn.sigmoid(jnp.dot(seg, wi16, precision=hp) + tw)
        z = jnp.dot(mu, w2_ref[...], precision=hp) / se
        outs.append(se * jnp.log1p(jnp.exp(z)))
    vf_ref[...] = jnp.concatenate([v_ref[...]] + outs, axis=1)


def _tc_node(huv0, huv1, t2, Wi, W2, bi2, se2, V):
    g = N // RT
    return pl.pallas_call(
        _node_body,
        grid=(g,),
        in_specs=[pl.BlockSpec((RT, DA), lambda i: (i, 0)),
                  pl.BlockSpec((RT, DA), lambda i: (i, 0)),
                  pl.BlockSpec((RT, 1), lambda i: (i, 0)),
                  pl.BlockSpec((HF + 1, D), lambda i: (0, 0)),
                  pl.BlockSpec((D, M), lambda i: (0, 0)),
                  pl.BlockSpec((1, D), lambda i: (0, 0)),
                  pl.BlockSpec((1, M), lambda i: (0, 0)),
                  pl.BlockSpec((RT, D), lambda i: (i, 0))],
        out_specs=[pl.BlockSpec((RT, D + H * M), lambda i: (i, 0)),
                   pl.BlockSpec((RT, D), lambda i: (i, 0))],
        out_shape=[jax.ShapeDtypeStruct((N, D + H * M), jnp.float32),
                   jax.ShapeDtypeStruct((N, D), jnp.float32)],
    )(huv0, huv1, t2, Wi, W2, bi2, se2, V)


def _final_body(o0_ref, o1_ref, rden_ref, xd_ref, out_ref):
    out_ref[...] = (o0_ref[...] + o1_ref[...]) * rden_ref[...] + xd_ref[...]


def _tc_final(ou0, ou1, rden, xd2):
    g = N // RT
    row = pl.BlockSpec((RT, D), lambda i: (i, 0))
    return pl.pallas_call(
        _final_body,
        grid=(g,),
        in_specs=[row, row, row, row],
        out_specs=row,
        out_shape=jax.ShapeDtypeStruct((N, D), jnp.float32),
    )(ou0, ou1, rden, xd2)


_MESH = plsc.VectorSubcoreMesh(core_axis_name="c", subcore_axis_name="s",
                               num_cores=NC, num_subcores=NS)
_SC_PARAMS = pltpu.CompilerParams(needs_layout_passes=False,
                                  use_tc_tiling_on_sc=False)
_ISCALE = float(1.0 / np.sqrt(HF))


def _zero_vmem(buf, rows, width):
    def zrow(r, _):
        for c in range(width // 16):
            buf[r, pl.ds(c * 16, 16)] = jnp.zeros((16,), jnp.float32)
        return 0
    lax.fori_loop(0, rows, zrow, 0)


def _zero_shared(shared, stage, sid):
    row0 = sid * ROWS_PER_TILE

    def zcp(k, _):
        pltpu.sync_copy(stage.at[pl.ds(0, ZR)],
                        shared.at[pl.ds(row0 + k * ZR, ZR)])
        return 0
    lax.fori_loop(0, ROWS_PER_TILE // ZR, zcp, 0)


def _flush_shared(shared, stage, out_hbm, cid, sid):
    def fl(k, _):
        r0 = sid * ROWS_PER_TILE + k * ZR
        pltpu.sync_copy(shared.at[pl.ds(r0, ZR)], stage.at[pl.ds(0, ZR)])
        pltpu.sync_copy(stage.at[pl.ds(0, ZR)], out_hbm.at[cid, pl.ds(r0, ZR)])
        return 0
    lax.fori_loop(0, ROWS_PER_TILE // ZR, fl, 0)


@functools.partial(
    pl.kernel,
    out_type=(jax.ShapeDtypeStruct((E, H), jnp.float32),
              jax.ShapeDtypeStruct((NC, N, DA), jnp.float32)),
    mesh=_MESH,
    compiler_params=_SC_PARAMS,
    scratch_types=[
        pltpu.VMEM((2, 3, B), jnp.int32),
        pltpu.VMEM((2, B), jnp.int32),
        pltpu.VMEM((2, B, DQ), jnp.float32),
        pltpu.VMEM((2, B, 2 * D), jnp.float32),
        pltpu.VMEM((2, B, DA), jnp.float32),
        pltpu.VMEM((2, B, H), jnp.float32),
        pltpu.VMEM_SHARED((N, DA), jnp.float32),
        pltpu.SemaphoreType.DMA,
        pltpu.SemaphoreType.DMA,
        pltpu.SemaphoreType.DMA,
        pltpu.SemaphoreType.DMA,
        pltpu.SemaphoreType.DMA,
        pltpu.SemaphoreType.DMA,
        pltpu.SemaphoreType.DMA,
        pltpu.SemaphoreType.DMA,
    ],
)
def _sc_pass1(q_hbm, kt_hbm, meta_hbm,
              a_hbm, huv_hbm,
              mbuf, didxs, qbuf, ktbuf, tabuf, abuf,
              hu_s,
              semm0, semm1, semg0, semg1, semsc0, semsc1, sema0, sema1):
    cid = lax.axis_index("c")
    sid = lax.axis_index("s")
    wid = cid * NS + sid
    semm = (semm0, semm1)
    semg = (semg0, semg1)
    semsc = (semsc0, semsc1)
    sema = (sema0, sema1)

    _zero_vmem(tabuf.at[0], B, DA)
    _zero_vmem(tabuf.at[1], B, DA)
    _zero_shared(hu_s, tabuf.at[0], sid)
    plsc.subcore_barrier()

    def blk_of(it):
        return wid + it * NW

    def issue(it, p):
        blk = blk_of(it)

        @pl.when(blk < BLOCKS)
        def _():
            mb = mbuf.at[p]
            pltpu.make_async_copy(meta_hbm.at[blk], mb, semm[p]).wait()
            pltpu.async_copy(q_hbm.at[mb.at[1]], qbuf.at[p], semg[p])
            pltpu.async_copy(kt_hbm.at[mb.at[0]], ktbuf.at[p], semg[p])

    def process(it, p):
        blk = blk_of(it)

        @pl.when(blk < BLOCKS)
        def _():
            mb = mbuf.at[p]
            tab = tabuf.at[p]
            ab = abuf.at[p]
            base = blk * B
            pltpu.make_async_copy(q_hbm.at[mb.at[1]], qbuf.at[p], semg[p]).wait()
            pltpu.make_async_copy(kt_hbm.at[mb.at[0]], ktbuf.at[p], semg[p]).wait()

            @pl.when(it >= 2)
            def _():
                pltpu.make_async_copy(tab, hu_s.at[didxs.at[p]], semsc[p]).wait()
                pltpu.make_async_copy(ab, a_hbm.at[pl.ds(base, B)], sema[p]).wait()

            for c in range(B // 16):
                didxs[p, pl.ds(c * 16, 16)] = mb[1, pl.ds(c * 16, 16)]

            def group(g, _):
                rot = lax.iota(jnp.int32, 16)
                rows = g * 16 + rot
                efv = plsc.bitcast(mb[2, pl.ds(g * 16, 16)], jnp.float32)
                qb = qbuf.at[p]
                kb = ktbuf.at[p]
                tb = ktbuf.at[p]
                def dot_step(f2, accs):
                    out = list(accs)
                    for u in range(2):
                        rc = jnp.bitwise_and(2 * f2 + u + rot, HF - 1)
                        for h in range(H):
                            col = rc + h * HF
                            qg = plsc.load_gather(qb, [rows, col])
                            kg = plsc.load_gather(kb, [rows, col])
                            out[h] = out[h] + kg * qg
                    return tuple(out)

                zero16 = jnp.zeros((16,), jnp.float32)
                accs = lax.fori_loop(0, HF // 2, dot_step, (zero16,) * H)
                ahs = []
                for h in range(H):
                    wqg = plsc.load_gather(qb, [rows, jnp.full((16,), D + h, jnp.int32)])
                    ah = jnp.exp((accs[h] + efv * wqg) * _ISCALE)
                    plsc.store_scatter(ab, [rows, jnp.full((16,), h, jnp.int32)], ah)
                    plsc.store_scatter(tab, [rows, jnp.full((16,), D + h, jnp.int32)], ah)
                    ahs.append(ah)

                def tsc_step(f2, c):
                    for u in range(2):
                        rc = jnp.bitwise_and(2 * f2 + u + rot, HF - 1)
                        for h in range(H):
                            col = rc + h * HF
                            tg = plsc.load_gather(tb, [rows, D + col])
                            plsc.store_scatter(tab, [rows, col], tg * ahs[h])
                    return c

                lax.fori_loop(0, HF // 2, tsc_step, 0)
                return 0

            lax.fori_loop(0, B // 16, group, 0)
            pltpu.async_copy(ab, a_hbm.at[pl.ds(base, B)], sema[p])
            pltpu.async_copy(tab, hu_s.at[didxs.at[p]], semsc[p], add=True)

            blk2 = blk + 2 * NW

            @pl.when(blk2 < BLOCKS)
            def _():
                pltpu.async_copy(meta_hbm.at[blk2], mb, semm[p])

    pltpu.sync_copy(meta_hbm.at[blk_of(0)], mbuf.at[0])
    pltpu.async_copy(meta_hbm.at[blk_of(1)], mbuf.at[1], semm1)
    pltpu.async_copy(q_hbm.at[mbuf.at[0].at[1]], qbuf.at[0], semg0)
    pltpu.async_copy(kt_hbm.at[mbuf.at[0].at[0]], ktbuf.at[0], semg0)

    def body2(k, _):
        issue(2 * k + 1, 1)
        process(2 * k, 0)
        issue(2 * k + 2, 0)
        process(2 * k + 1, 1)
        return 0

    lax.fori_loop(0, MAXI2, body2, 0)

    pltpu.make_async_copy(tabuf.at[0], hu_s.at[didxs.at[0]], semsc0).wait()
    pltpu.make_async_copy(abuf.at[0], a_hbm.at[pl.ds(0, B)], sema0).wait()
    pltpu.make_async_copy(tabuf.at[1], hu_s.at[didxs.at[1]], semsc1).wait()
    pltpu.make_async_copy(abuf.at[1], a_hbm.at[pl.ds(0, B)], sema1).wait()
    plsc.subcore_barrier()
    _flush_shared(hu_s, tabuf.at[0], huv_hbm, cid, sid)


@functools.partial(
    pl.kernel,
    out_type=jax.ShapeDtypeStruct((NC, N, D), jnp.float32),
    mesh=_MESH,
    compiler_params=_SC_PARAMS,
    scratch_types=[
        pltpu.VMEM((2, 3, B), jnp.int32),
        pltpu.VMEM((2, B), jnp.int32),
        pltpu.VMEM((2, B, D + H * M), jnp.float32),
        pltpu.VMEM((2, B, H * M), jnp.float32),
        pltpu.VMEM((2, B, D), jnp.float32),
        pltpu.VMEM((2, B, H), jnp.float32),
        pltpu.VMEM_SHARED((N, D), jnp.float32),
        pltpu.SemaphoreType.DMA,
        pltpu.SemaphoreType.DMA,
        pltpu.SemaphoreType.DMA,
        pltpu.SemaphoreType.DMA,
        pltpu.SemaphoreType.DMA,
        pltpu.SemaphoreType.DMA,
    ],
)
def _sc_pass2(vf_hbm, m_hbm, a_hbm, meta_hbm,
              ou_hbm,
              mbuf, didxs, vfbuf, m2buf, obuf, ain,
              ou_s,
              semm0, semm1, semg0, semg1, semsc0, semsc1):
    cid = lax.axis_index("c")
    sid = lax.axis_index("s")
    wid = cid * NS + sid
    semm = (semm0, semm1)
    semg = (semg0, semg1)
    semsc = (semsc0, semsc1)

    _zero_vmem(obuf.at[0], B, D)
    _zero_shared(ou_s, obuf.at[0], sid)
    plsc.subcore_barrier()

    def blk_of(it):
        return wid + it * NW

    def issue(it, p):
        blk = blk_of(it)

        @pl.when(blk < BLOCKS)
        def _():
            mb = mbuf.at[p]
            base = blk * B
            pltpu.make_async_copy(meta_hbm.at[blk], mb, semm[p]).wait()
            pltpu.async_copy(vf_hbm.at[mb.at[0]], vfbuf.at[p], semg[p])
            pltpu.async_copy(m_hbm.at[mb.at[1]], m2buf.at[p], semg[p])
            pltpu.async_copy(a_hbm.at[pl.ds(base, B)], ain.at[p], semg[p])

    def process(it, p):
        blk = blk_of(it)

        @pl.when(blk < BLOCKS)
        def _():
            mb = mbuf.at[p]
            ob = obuf.at[p]
            base = blk * B
            pltpu.make_async_copy(vf_hbm.at[mb.at[0]], vfbuf.at[p], semg[p]).wait()
            pltpu.make_async_copy(m_hbm.at[mb.at[1]], m2buf.at[p], semg[p]).wait()
            pltpu.make_async_copy(a_hbm.at[pl.ds(base, B)], ain.at[p], semg[p]).wait()

            @pl.when(it >= 2)
            def _():
                pltpu.make_async_copy(ob, ou_s.at[didxs.at[p]], semsc[p]).wait()

            for c in range(B // 16):
                didxs[p, pl.ds(c * 16, 16)] = mb[1, pl.ds(c * 16, 16)]

            def group(g, _):
                rot = lax.iota(jnp.int32, 16)
                rows = g * 16 + rot
                fb = vfbuf.at[p]
                m2 = m2buf.at[p]
                vb = vfbuf.at[p]
                ai = ain.at[p]
                def dot_step(j2, accs):
                    out = list(accs)
                    for u in range(2):
                        rc = jnp.bitwise_and(2 * j2 + u + rot, M - 1)
                        for h in range(H):
                            col = rc + h * M
                            fg = plsc.load_gather(fb, [rows, D + col])
                            mg = plsc.load_gather(m2, [rows, col])
                            out[h] = out[h] + fg * mg
                    return tuple(out)

                zero16 = jnp.zeros((16,), jnp.float32)
                accs = lax.fori_loop(0, M // 2, dot_step, (zero16,) * H)
                ees = []
                for h in range(H):
                    ag = plsc.load_gather(ai, [rows, jnp.full((16,), h, jnp.int32)])
                    ees.append(accs[h] * ag)

                def vsc_step(f2, c):
                    for u in range(2):
                        rc = jnp.bitwise_and(2 * f2 + u + rot, HF - 1)
                        for h in range(H):
                            col = rc + h * HF
                            vg = plsc.load_gather(vb, [rows, col])
                            plsc.store_scatter(ob, [rows, col], vg * ees[h])
                    return c

                lax.fori_loop(0, HF // 2, vsc_step, 0)
                return 0

            lax.fori_loop(0, B // 16, group, 0)
            pltpu.async_copy(ob, ou_s.at[didxs.at[p]], semsc[p], add=True)

            blk2 = blk + 2 * NW

            @pl.when(blk2 < BLOCKS)
            def _():
                pltpu.async_copy(meta_hbm.at[blk2], mb, semm[p])

    pltpu.sync_copy(meta_hbm.at[blk_of(0)], mbuf.at[0])
    pltpu.async_copy(meta_hbm.at[blk_of(1)], mbuf.at[1], semm1)
    pltpu.async_copy(vf_hbm.at[mbuf.at[0].at[0]], vfbuf.at[0], semg0)
    pltpu.async_copy(m_hbm.at[mbuf.at[0].at[1]], m2buf.at[0], semg0)
    pltpu.async_copy(a_hbm.at[pl.ds(wid * B, B)], ain.at[0], semg0)

    def body2(k, _):
        issue(2 * k + 1, 1)
        process(2 * k, 0)
        issue(2 * k + 2, 0)
        process(2 * k + 1, 1)
        return 0

    lax.fori_loop(0, MAXI2, body2, 0)

    pltpu.make_async_copy(obuf.at[0], ou_s.at[didxs.at[0]], semsc0).wait()
    pltpu.make_async_copy(obuf.at[1], ou_s.at[didxs.at[1]], semsc1).wait()
    plsc.subcore_barrier()
    _flush_shared(ou_s, obuf.at[0], ou_hbm, cid, sid)


def kernel(x_src, x_dst, t, m, ef, Wq, Wk, Wv, Wt, We, Wi, bi,
           weight_i, scale_i, edge_index):
    xs2 = x_src.reshape(N, D)
    xd2 = x_dst.reshape(N, D)
    src = edge_index[0]
    dst = edge_index[1]
    ef_bits = lax.bitcast_convert_type(ef.reshape(E), jnp.int32)

    meta = jnp.stack([src.reshape(BLOCKS, B), dst.reshape(BLOCKS, B),
                      ef_bits.reshape(BLOCKS, B)], axis=1)

    Qa, KT, V = _tc_proj(xs2, xd2, Wq, Wk, Wt, Wv, We)
    a_e, huv = _sc_pass1(Qa, KT, meta)

    flat = weight_i.reshape(HF * M)
    sel = jnp.arange(D)[:, None] // HF == jnp.arange(M)[None, :]
    W2 = jnp.where(sel, flat[:, None], 0.0)
    vf, rden = _tc_node(huv[0], huv[1], t.reshape(N, 1), Wi, W2,
                        bi.reshape(1, HF * M), scale_i.reshape(1, M), V)

    ou = _sc_pass2(vf, m.reshape(N, H * M), a_e, meta)
    out2 = _tc_final(ou[0], ou[1], rden, xd2)
    return out2.reshape(N, 1, D)
